# baseline (speedup 1.0000x reference)
import jax
import jax.numpy as jnp
from jax import lax
from jax.experimental import pallas as pl
from jax.experimental.pallas import tpu as pltpu

_CH = 200
_NBUF = 5


def _body(lap_ref, fbf_ref, w1t_ref, w2t_ref, b_ref, out_ref,
          fk_ref, *scr):
    bufs = scr[:_NBUF]
    obufs = scr[_NBUF:2 * _NBUF]
    isems = scr[2 * _NBUF]
    osems = scr[2 * _NBUF + 1]
    fsem = scr[2 * _NBUF + 2]
    n = lap_ref.shape[0]
    nchunk = n // _CH
    ngroups = nchunk // _NBUF

    def fill(chunk, j, sem):
        return pltpu.make_async_copy(
            lap_ref.at[pl.ds(chunk * _CH, _CH)], bufs[j], sem)

    def flush(chunk, j, sem):
        return pltpu.make_async_copy(
            obufs[j], out_ref.at[pl.ds(chunk * _CH, _CH)], sem)

    for j in range(_NBUF):
        fill(j, j, isems.at[j]).start()
    pltpu.make_async_copy(fbf_ref, fk_ref, fsem).start()
    pltpu.make_async_copy(fbf_ref, fk_ref, fsem).wait()
    fk = fk_ref[...]
    w1t = w1t_ref[...]
    w2t = w2t_ref[...]
    b = b_ref[...]

    def group(g, carry):
        for j in range(_NBUF):
            i = g * _NBUF + j
            fill(i, j, isems.at[j]).wait()
            x = jnp.dot(bufs[j][...].astype(jnp.bfloat16), fk,
                        preferred_element_type=jnp.float32)
            pj = (j - 1) % _NBUF
            nxt = i + _NBUF - 1
            @pl.when((i >= 1) & (nxt < nchunk))
            def _():
                fill(nxt, pj, isems.at[pj]).start()
            f = fk_ref[pl.ds(i * _CH, _CH), :]
            res = (
                jnp.dot((f + x).astype(jnp.bfloat16), w1t,
                        preferred_element_type=jnp.float32)
                + jnp.dot((x * f).astype(jnp.bfloat16), w2t,
                          preferred_element_type=jnp.float32)
                + b
            )
            @pl.when(g > 0)
            def _():
                flush(0, j, osems.at[j]).wait()
            obufs[j][...] = res
            flush(i, j, osems.at[j]).start()
        return carry

    lax.fori_loop(0, ngroups, group, 0)
    for j in range(_NBUF):
        flush(0, j, osems.at[j]).wait()


def kernel(lap_matrix, eye_matrix, features, W1, b1, W2, b2):
    del eye_matrix
    n, d = features.shape

    feat_bf = features.astype(jnp.bfloat16)
    w1t = W1.T.astype(jnp.bfloat16)
    w2t = W2.T.astype(jnp.bfloat16)
    bias = (b1 + b2).reshape(1, d)

    return pl.pallas_call(
        _body,
        in_specs=[
            pl.BlockSpec(memory_space=pl.ANY),
            pl.BlockSpec(memory_space=pl.ANY),
            pl.BlockSpec(memory_space=pltpu.VMEM),
            pl.BlockSpec(memory_space=pltpu.VMEM),
            pl.BlockSpec(memory_space=pltpu.VMEM),
        ],
        out_specs=pl.BlockSpec(memory_space=pl.ANY),
        out_shape=jax.ShapeDtypeStruct((n, d), jnp.float32),
        scratch_shapes=(
            [pltpu.VMEM((n, d), jnp.bfloat16)]
            + [pltpu.VMEM((_CH, n), jnp.float32) for _ in range(_NBUF)]
            + [pltpu.VMEM((_CH, d), jnp.float32) for _ in range(_NBUF)]
            + [pltpu.SemaphoreType.DMA((_NBUF,)),
               pltpu.SemaphoreType.DMA((_NBUF,)),
               pltpu.SemaphoreType.DMA]
        ),
    )(lap_matrix, feat_bf, w1t, w2t, bias)
